# Initial kernel scaffold; baseline (speedup 1.0000x reference)
import jax
import jax.numpy as jnp
from jax.experimental import pallas as pl


def kernel(x):
    raise NotImplementedError("write your pallas kernel here")


# speedup vs baseline: 9.6249x; 9.6249x over previous
import math

import jax
import jax.numpy as jnp
import numpy as np
from jax import lax
from jax.experimental import pallas as pl
from jax.experimental.pallas import tpu as pltpu

_SR = 16000
_HOP = 256
_DECAY = 0.5
_C = 256


def _norm_alpha():
    dt = _HOP / _SR
    a_ = math.exp(-dt / _DECAY)
    precision = 3
    a = 1.0
    while a >= 1.0:
        a = round(a_, precision)
        precision += 1
    return a


_ALPHA = _norm_alpha()


def _chunk_mats():
    s = np.arange(_C)[:, None].astype(np.float64)
    t = np.arange(_C)[None, :].astype(np.float64)
    d = t - s
    u = np.where(d >= 0.0, (1.0 - _ALPHA) * np.power(_ALPHA, np.maximum(d, 0.0)), 0.0)
    m = np.zeros((_C, _C), dtype=np.float64)
    m[_C - 1, :] = np.power(_ALPHA, np.arange(1, _C + 1, dtype=np.float64))
    return (jnp.asarray(u, dtype=jnp.bfloat16), jnp.asarray(m, dtype=jnp.bfloat16))


def _body(x_ref, u_ref, m_ref, o_ref, s_ref, *, F, T):
    t_idx = pl.program_id(1)

    @pl.when(t_idx == 0)
    def _init():
        f = lax.broadcasted_iota(jnp.int32, (F, _C), 0).astype(jnp.float32)
        init = 0.001 + f * ((0.0001 - 0.001) / (F - 1))
        s_ref[...] = init.astype(jnp.bfloat16)

    x = x_ref[0]
    xa = jnp.sqrt(jnp.maximum(x, 1e-10))
    col = lax.broadcasted_iota(jnp.int32, (F, _C), 1)
    valid = (t_idx * _C + col) < T
    xa = jnp.where(valid, xa, 0.0)

    s_new = jnp.dot(
        xa.astype(jnp.bfloat16), u_ref[...], preferred_element_type=jnp.float32
    ) + jnp.dot(s_ref[...], m_ref[...], preferred_element_type=jnp.float32)
    s_ref[...] = s_new.astype(jnp.bfloat16)
    o_ref[0] = x * lax.rsqrt(s_new)


@jax.jit
def kernel(x):
    B, F, T = x.shape
    n_t = (T + _C - 1) // _C
    u, m = _chunk_mats()
    import functools

    body = functools.partial(_body, F=F, T=T)
    return pl.pallas_call(
        body,
        grid=(B, n_t),
        in_specs=[
            pl.BlockSpec((1, F, _C), lambda b, t: (b, 0, t)),
            pl.BlockSpec((_C, _C), lambda b, t: (0, 0)),
            pl.BlockSpec((_C, _C), lambda b, t: (0, 0)),
        ],
        out_specs=pl.BlockSpec((1, F, _C), lambda b, t: (b, 0, t)),
        out_shape=jax.ShapeDtypeStruct((B, F, T), x.dtype),
        scratch_shapes=[pltpu.VMEM((F, _C), jnp.bfloat16)],
        compiler_params=pltpu.CompilerParams(
            dimension_semantics=("parallel", "arbitrary"),
        ),
    )(x, u, m)


# speedup vs baseline: 14.1673x; 1.4719x over previous
import math

import jax
import jax.numpy as jnp
import numpy as np
from jax import lax
from jax.experimental import pallas as pl
from jax.experimental.pallas import tpu as pltpu

_SR = 16000
_HOP = 256
_DECAY = 0.5
_C = 256


def _norm_alpha():
    dt = _HOP / _SR
    a_ = math.exp(-dt / _DECAY)
    precision = 3
    a = 1.0
    while a >= 1.0:
        a = round(a_, precision)
        precision += 1
    return a


_ALPHA = _norm_alpha()


def _chunk_mats():
    s = np.arange(_C)[:, None].astype(np.float64)
    t = np.arange(_C)[None, :].astype(np.float64)
    d = t - s
    u = np.where(d >= 0.0, (1.0 - _ALPHA) * np.power(_ALPHA, np.maximum(d, 0.0)), 0.0)
    m = np.zeros((_C, _C), dtype=np.float64)
    m[_C - 1, :] = np.power(_ALPHA, np.arange(1, _C + 1, dtype=np.float64))
    return (jnp.asarray(u, dtype=jnp.bfloat16), jnp.asarray(m, dtype=jnp.bfloat16))


_G = 8


def _body(x_ref, u_ref, m_ref, o_ref, s_ref, *, F, T):
    t_idx = pl.program_id(1)

    @pl.when(t_idx == 0)
    def _init():
        f = lax.broadcasted_iota(jnp.int32, (F, _C), 0).astype(jnp.float32)
        init = (0.001 + f * ((0.0001 - 0.001) / (F - 1))).astype(jnp.bfloat16)
        for g in range(_G):
            s_ref[g] = init

    col = lax.broadcasted_iota(jnp.int32, (F, _C), 1)
    valid = (t_idx * _C + col) < T
    u = u_ref[...]
    m = m_ref[...]
    for g in range(_G):
        x = x_ref[g]
        y = jnp.maximum(x, 1e-10)
        xa = jnp.where(valid, y * lax.rsqrt(y), 0.0)
        s_new = jnp.dot(
            xa.astype(jnp.bfloat16), u, preferred_element_type=jnp.float32
        ) + jnp.dot(s_ref[g], m, preferred_element_type=jnp.float32)
        s_ref[g] = s_new.astype(jnp.bfloat16)
        o_ref[g] = x * lax.rsqrt(s_new)


@jax.jit
def kernel(x):
    B, F, T = x.shape
    n_t = (T + _C - 1) // _C
    u, m = _chunk_mats()
    import functools

    body = functools.partial(_body, F=F, T=T)
    return pl.pallas_call(
        body,
        grid=(B // _G, n_t),
        in_specs=[
            pl.BlockSpec((_G, F, _C), lambda b, t: (b, 0, t)),
            pl.BlockSpec((_C, _C), lambda b, t: (0, 0)),
            pl.BlockSpec((_C, _C), lambda b, t: (0, 0)),
        ],
        out_specs=pl.BlockSpec((_G, F, _C), lambda b, t: (b, 0, t)),
        out_shape=jax.ShapeDtypeStruct((B, F, T), x.dtype),
        scratch_shapes=[pltpu.VMEM((_G, F, _C), jnp.bfloat16)],
        compiler_params=pltpu.CompilerParams(
            dimension_semantics=("parallel", "arbitrary"),
        ),
    )(x, u, m)


# speedup vs baseline: 14.1764x; 1.0006x over previous
import math

import jax
import jax.numpy as jnp
import numpy as np
from jax import lax
from jax.experimental import pallas as pl
from jax.experimental.pallas import tpu as pltpu

_SR = 16000
_HOP = 256
_DECAY = 0.5
_C = 256


def _norm_alpha():
    dt = _HOP / _SR
    a_ = math.exp(-dt / _DECAY)
    precision = 3
    a = 1.0
    while a >= 1.0:
        a = round(a_, precision)
        precision += 1
    return a


_ALPHA = _norm_alpha()


def _chunk_mats():
    s = np.arange(_C)[:, None].astype(np.float64)
    t = np.arange(_C)[None, :].astype(np.float64)
    d = t - s
    u = np.where(d >= 0.0, (1.0 - _ALPHA) * np.power(_ALPHA, np.maximum(d, 0.0)), 0.0)
    m = np.zeros((_C, _C), dtype=np.float64)
    m[_C - 1, :] = np.power(_ALPHA, np.arange(1, _C + 1, dtype=np.float64))
    return (jnp.asarray(u, dtype=jnp.bfloat16), jnp.asarray(m, dtype=jnp.bfloat16))


_G = 2
_TB = 1024


def _body(x_ref, u_ref, m_ref, o_ref, s_ref, *, F, T):
    t_idx = pl.program_id(1)

    @pl.when(t_idx == 0)
    def _init():
        f = lax.broadcasted_iota(jnp.int32, (F, _C), 0).astype(jnp.float32)
        init = (0.001 + f * ((0.0001 - 0.001) / (F - 1))).astype(jnp.bfloat16)
        for g in range(_G):
            s_ref[g] = init

    col = lax.broadcasted_iota(jnp.int32, (F, _C), 1)
    u = u_ref[...]
    m = m_ref[...]
    for tc in range(_TB // _C):
        valid = (t_idx * _TB + tc * _C + col) < T
        for g in range(_G):
            x = x_ref[g, :, tc * _C:(tc + 1) * _C]
            y = jnp.maximum(x, 1e-10)
            xa = jnp.where(valid, y * lax.rsqrt(y), 0.0)
            s_new = jnp.dot(
                xa.astype(jnp.bfloat16), u, preferred_element_type=jnp.float32
            ) + jnp.dot(s_ref[g], m, preferred_element_type=jnp.float32)
            s_ref[g] = s_new.astype(jnp.bfloat16)
            o_ref[g, :, tc * _C:(tc + 1) * _C] = x * lax.rsqrt(s_new)


@jax.jit
def kernel(x):
    B, F, T = x.shape
    n_t = (T + _TB - 1) // _TB
    u, m = _chunk_mats()
    import functools

    body = functools.partial(_body, F=F, T=T)
    return pl.pallas_call(
        body,
        grid=(B // _G, n_t),
        in_specs=[
            pl.BlockSpec((_G, F, _TB), lambda b, t: (b, 0, t)),
            pl.BlockSpec((_C, _C), lambda b, t: (0, 0)),
            pl.BlockSpec((_C, _C), lambda b, t: (0, 0)),
        ],
        out_specs=pl.BlockSpec((_G, F, _TB), lambda b, t: (b, 0, t)),
        out_shape=jax.ShapeDtypeStruct((B, F, T), x.dtype),
        scratch_shapes=[pltpu.VMEM((_G, F, _C), jnp.bfloat16)],
        compiler_params=pltpu.CompilerParams(
            dimension_semantics=("parallel", "arbitrary"),
        ),
    )(x, u, m)


# speedup vs baseline: 14.2841x; 1.0076x over previous
import math

import jax
import jax.numpy as jnp
import numpy as np
from jax import lax
from jax.experimental import pallas as pl
from jax.experimental.pallas import tpu as pltpu

_SR = 16000
_HOP = 256
_DECAY = 0.5
_C = 256


def _norm_alpha():
    dt = _HOP / _SR
    a_ = math.exp(-dt / _DECAY)
    precision = 3
    a = 1.0
    while a >= 1.0:
        a = round(a_, precision)
        precision += 1
    return a


_ALPHA = _norm_alpha()


def _chunk_mats():
    s = np.arange(_C)[:, None].astype(np.float64)
    t = np.arange(_C)[None, :].astype(np.float64)
    d = t - s
    u = np.where(d >= 0.0, (1.0 - _ALPHA) * np.power(_ALPHA, np.maximum(d, 0.0)), 0.0)
    m = np.zeros((_C, _C), dtype=np.float64)
    m[_C - 1, :] = np.power(_ALPHA, np.arange(1, _C + 1, dtype=np.float64))
    return (jnp.asarray(u, dtype=jnp.bfloat16), jnp.asarray(m, dtype=jnp.bfloat16))


_G = 2
_TB = 2048


def _body(x_ref, u_ref, m_ref, o_ref, s_ref, *, F, T):
    t_idx = pl.program_id(1)

    @pl.when(t_idx == 0)
    def _init():
        f = lax.broadcasted_iota(jnp.int32, (F, _C), 0).astype(jnp.float32)
        init = (0.001 + f * ((0.0001 - 0.001) / (F - 1))).astype(jnp.bfloat16)
        for g in range(_G):
            s_ref[g] = init

    col = lax.broadcasted_iota(jnp.int32, (F, _C), 1)
    u = u_ref[...]
    m = m_ref[...]
    for tc in range(_TB // _C):
        valid = (t_idx * _TB + tc * _C + col) < T
        for g in range(_G):
            x = x_ref[g, :, tc * _C:(tc + 1) * _C]
            y = jnp.maximum(x, 1e-10)
            xa = jnp.where(valid, y * lax.rsqrt(y), 0.0)
            s_new = jnp.dot(
                xa.astype(jnp.bfloat16), u, preferred_element_type=jnp.float32
            ) + jnp.dot(s_ref[g], m, preferred_element_type=jnp.float32)
            s_ref[g] = s_new.astype(jnp.bfloat16)
            o_ref[g, :, tc * _C:(tc + 1) * _C] = x * lax.rsqrt(s_new)


@jax.jit
def kernel(x):
    B, F, T = x.shape
    n_t = (T + _TB - 1) // _TB
    u, m = _chunk_mats()
    import functools

    body = functools.partial(_body, F=F, T=T)
    return pl.pallas_call(
        body,
        grid=(B // _G, n_t),
        in_specs=[
            pl.BlockSpec((_G, F, _TB), lambda b, t: (b, 0, t)),
            pl.BlockSpec((_C, _C), lambda b, t: (0, 0)),
            pl.BlockSpec((_C, _C), lambda b, t: (0, 0)),
        ],
        out_specs=pl.BlockSpec((_G, F, _TB), lambda b, t: (b, 0, t)),
        out_shape=jax.ShapeDtypeStruct((B, F, T), x.dtype),
        scratch_shapes=[pltpu.VMEM((_G, F, _C), jnp.bfloat16)],
        compiler_params=pltpu.CompilerParams(
            dimension_semantics=("parallel", "arbitrary"),
            vmem_limit_bytes=48 * 1024 * 1024,
        ),
    )(x, u, m)
